# Initial kernel scaffold; baseline (speedup 1.0000x reference)
import jax
import jax.numpy as jnp
from jax.experimental import pallas as pl


def kernel(x, W):
    raise NotImplementedError("write your pallas kernel here")


# speedup vs baseline: 3.2900x; 3.2900x over previous
import jax
import jax.numpy as jnp
import numpy as np
from jax import lax
from jax.experimental import pallas as pl
from jax.experimental.pallas import tpu as pltpu
from jax.experimental.pallas import tpu_sc as plsc

AA = 20
MP = 15
F = 3
NP = 34
LOUT = AA + F - 1
XC = 1 + MP * AA + NP
OC = NP * LOUT
LANES = 16
NWORK = 32

_P2J = {0: [0], 1: [1, 2], 2: [0, 1], 3: [2], 4: [1], 6: [2, 3], 7: [3],
        10: [4], 12: [5], 14: [6, 7], 15: [7], 17: [8], 18: [5, 6], 19: [7],
        21: [8], 22: [7, 8], 24: [8], 25: [6], 27: [4], 28: [3], 30: [2],
        31: [1], 33: [0]}
_PEP_POS = sorted({j for js in _P2J.values() for j in js})


def _body(xh, wh, oh, wv, xv, ov, nblk):
    cid = lax.axis_index("c")
    sid = lax.axis_index("s")
    wid = sid * 2 + cid

    pltpu.sync_copy(wh, wv)

    zero = jnp.zeros((LANES,), jnp.float32)

    def zb(i, c):
        ov[pl.ds(i * LANES, LANES)] = zero
        return c

    lax.fori_loop(0, OC, zb, 0)

    lane = lax.iota(jnp.int32, LANES)
    ibase = lane * XC
    obase = lane * OC

    def blk(i, c):
        base = (wid * nblk + i) * LANES
        pltpu.sync_copy(xh.at[pl.ds(base * XC, LANES * XC)], xv)
        pep = {}
        for j in _PEP_POS:
            pep[j] = [plsc.load_gather(xv, [ibase + (1 + j * AA + a)])
                      for a in range(AA)]
        aggs = {}
        for js in _P2J.values():
            key = tuple(js)
            if key in aggs:
                continue
            if len(js) == 1:
                aggs[key] = pep[js[0]]
            else:
                aggs[key] = [pep[js[0]][a] + pep[js[1]][a] for a in range(AA)]
        for p, js in _P2J.items():
            agg = aggs[tuple(js)]
            pidx = plsc.load_gather(xv, [ibase + (1 + MP * AA + p)])
            fb = pidx.astype(jnp.int32) * F
            f = [plsc.load_gather(wv, [fb + t]) for t in range(F)]
            col = p * LOUT
            for l in range(LOUT):
                acc = None
                for t in range(F):
                    a = l - t
                    if 0 <= a < AA:
                        term = f[t] * agg[a]
                        acc = term if acc is None else acc + term
                plsc.store_scatter(ov, [obase + (col + l)], acc)
        pltpu.sync_copy(ov, oh.at[pl.ds(base * OC, LANES * OC)])
        return c

    lax.fori_loop(0, nblk, blk, 0)


def kernel(x, W):
    B = x.shape[0]
    nblk = B // (NWORK * LANES)
    mesh = plsc.VectorSubcoreMesh(core_axis_name="c", subcore_axis_name="s")
    run = pl.kernel(
        lambda xh, wh, oh, wv, xv, ov: _body(xh, wh, oh, wv, xv, ov, nblk),
        out_type=jax.ShapeDtypeStruct((B * OC,), jnp.float32),
        mesh=mesh,
        scratch_types=[
            pltpu.VMEM((AA * F,), jnp.float32),
            pltpu.VMEM((LANES * XC,), jnp.float32),
            pltpu.VMEM((LANES * OC,), jnp.float32),
        ],
        compiler_params=pltpu.CompilerParams(needs_layout_passes=False),
    )
    out = run(x.reshape(-1), W.reshape(-1))
    return out.reshape(B, NP, LOUT)


# speedup vs baseline: 17.4661x; 5.3088x over previous
import jax
import jax.numpy as jnp
import numpy as np
from jax import lax
from jax.experimental import pallas as pl
from jax.experimental.pallas import tpu as pltpu
from jax.experimental.pallas import tpu_sc as plsc

AA = 20
MP = 15
F = 3
NP = 34
LOUT = AA + F - 1
XC = 1 + MP * AA + NP
LANES = 16
NWORK = 32
BLK = 64
NCH = BLK // LANES
NPEP = 9 * AA

_P2J = {0: [0], 1: [1, 2], 2: [0, 1], 3: [2], 4: [1], 6: [2, 3], 7: [3],
        10: [4], 12: [5], 14: [6, 7], 15: [7], 17: [8], 18: [5, 6], 19: [7],
        21: [8], 22: [7, 8], 24: [8], 25: [6], 27: [4], 28: [3], 30: [2],
        31: [1], 33: [0]}
_INACTIVE = [p for p in range(NP) if p not in _P2J]


def _body(xh, wh, oh, wv, xa, xp, ov, nblk):
    cid = lax.axis_index("c")
    sid = lax.axis_index("s")
    wid = sid * 2 + cid

    pltpu.sync_copy(wh, wv)

    zero = jnp.zeros((LANES,), jnp.float32)

    def zc(c, carry):
        for p in _INACTIVE:
            for l in range(LOUT):
                ov[p, l, pl.ds(c * LANES, LANES)] = zero
        return carry

    lax.fori_loop(0, NCH, zc, 0)

    def blk(i, carry):
        base = (wid * nblk + i) * BLK
        pltpu.sync_copy(xh.at[pl.ds(1, NPEP), pl.ds(base, BLK)], xa)
        pltpu.sync_copy(xh.at[pl.ds(1 + MP * AA, NP), pl.ds(base, BLK)], xp)

        def chunk(c, carry2):
            off = c * LANES
            pep = {}
            for j in range(9):
                pep[j] = [xa[j * AA + a, pl.ds(off, LANES)] for a in range(AA)]
            aggs = {}
            for js in _P2J.values():
                key = tuple(js)
                if key in aggs:
                    continue
                if len(js) == 1:
                    aggs[key] = pep[js[0]]
                else:
                    aggs[key] = [pep[js[0]][a] + pep[js[1]][a]
                                 for a in range(AA)]
            for p, js in _P2J.items():
                agg = aggs[tuple(js)]
                fb = xp[p, pl.ds(off, LANES)].astype(jnp.int32) * F
                f = [plsc.load_gather(wv, [fb + t]) for t in range(F)]
                for l in range(LOUT):
                    acc = None
                    for t in range(F):
                        a = l - t
                        if 0 <= a < AA:
                            term = f[t] * agg[a]
                            acc = term if acc is None else acc + term
                    ov[p, l, pl.ds(off, LANES)] = acc
            return carry2

        lax.fori_loop(0, NCH, chunk, 0)
        pltpu.sync_copy(ov, oh.at[:, :, pl.ds(base, BLK)])
        return carry

    lax.fori_loop(0, nblk, blk, 0)


def kernel(x, W):
    B = x.shape[0]
    nblk = B // (NWORK * BLK)
    mesh = plsc.VectorSubcoreMesh(core_axis_name="c", subcore_axis_name="s")
    run = pl.kernel(
        lambda xh, wh, oh, wv, xa, xp, ov: _body(xh, wh, oh, wv, xa, xp, ov,
                                                 nblk),
        out_type=jax.ShapeDtypeStruct((NP, LOUT, B), jnp.float32),
        mesh=mesh,
        scratch_types=[
            pltpu.VMEM((AA * F,), jnp.float32),
            pltpu.VMEM((NPEP, BLK), jnp.float32),
            pltpu.VMEM((NP, BLK), jnp.float32),
            pltpu.VMEM((NP, LOUT, BLK), jnp.float32),
        ],
        compiler_params=pltpu.CompilerParams(needs_layout_passes=False,
                                             use_tc_tiling_on_sc=False),
    )
    out = run(x.T, W.reshape(-1))
    return jnp.transpose(out, (2, 0, 1))


# speedup vs baseline: 36.3184x; 2.0794x over previous
import jax
import jax.numpy as jnp
import numpy as np
from jax import lax
from jax.experimental import pallas as pl
from jax.experimental.pallas import tpu as pltpu
from jax.experimental.pallas import tpu_sc as plsc

AA = 20
MP = 15
F = 3
NP = 34
LOUT = AA + F - 1
XC = 1 + MP * AA + NP
LANES = 16
NWORK = 32
BLK = 128
NCH = BLK // LANES
XA_ROWS = 184
XP_OFF = 296
XP_ROWS = 39
PG = 17

_P2J = {0: [0], 1: [1, 2], 2: [0, 1], 3: [2], 4: [1], 6: [2, 3], 7: [3],
        10: [4], 12: [5], 14: [6, 7], 15: [7], 17: [8], 18: [5, 6], 19: [7],
        21: [8], 22: [7, 8], 24: [8], 25: [6], 27: [4], 28: [3], 30: [2],
        31: [1], 33: [0]}


def _body(xh, wh, oh, wv, xa, xp, ov, nblk):
    cid = lax.axis_index("c")
    sid = lax.axis_index("s")
    wid = sid * 2 + cid

    pltpu.sync_copy(wh, wv)

    zero = jnp.zeros((LANES,), jnp.float32)

    def blk(i, carry):
        base = (wid * nblk + i) * BLK
        pltpu.sync_copy(xh.at[pl.ds(0, XA_ROWS), pl.ds(base, BLK)], xa)
        pltpu.sync_copy(xh.at[pl.ds(XP_OFF, XP_ROWS), pl.ds(base, BLK)], xp)

        for p0 in (0, PG):
            def chunk(c, carry2, p0=p0):
                off = c * LANES
                pep = {}

                def pvec(j, a):
                    if (j, a) not in pep:
                        pep[(j, a)] = xa[1 + j * AA + a, pl.ds(off, LANES)]
                    return pep[(j, a)]

                aggs = {}
                for p in range(p0, p0 + PG):
                    js = _P2J.get(p)
                    if js is None or tuple(js) in aggs:
                        continue
                    if len(js) == 1:
                        aggs[tuple(js)] = [pvec(js[0], a) for a in range(AA)]
                    else:
                        aggs[tuple(js)] = [pvec(js[0], a) + pvec(js[1], a)
                                           for a in range(AA)]
                for p in range(p0, p0 + PG):
                    js = _P2J.get(p)
                    if js is None:
                        for l in range(LOUT):
                            ov[p - p0, l, pl.ds(off, LANES)] = zero
                        continue
                    agg = aggs[tuple(js)]
                    fb = (xp[301 - XP_OFF + p, pl.ds(off, LANES)]
                          .astype(jnp.int32) * F)
                    f = [plsc.load_gather(wv, [fb + t]) for t in range(F)]
                    for l in range(LOUT):
                        acc = None
                        for t in range(F):
                            a = l - t
                            if 0 <= a < AA:
                                term = f[t] * agg[a]
                                acc = term if acc is None else acc + term
                        ov[p - p0, l, pl.ds(off, LANES)] = acc
                return carry2

            lax.fori_loop(0, NCH, chunk, 0)
            pltpu.sync_copy(ov.at[:, pl.ds(0, LOUT), :],
                            oh.at[pl.ds(p0, PG), :, pl.ds(base, BLK)])
        return carry

    lax.fori_loop(0, nblk, blk, 0)


def kernel(x, W):
    B = x.shape[0]
    nblk = B // (NWORK * BLK)
    mesh = plsc.VectorSubcoreMesh(core_axis_name="c", subcore_axis_name="s")
    run = pl.kernel(
        lambda xh, wh, oh, wv, xa, xp, ov: _body(xh, wh, oh, wv, xa, xp, ov,
                                                 nblk),
        out_type=jax.ShapeDtypeStruct((NP, LOUT, B), jnp.float32),
        mesh=mesh,
        scratch_types=[
            pltpu.VMEM((AA * F,), jnp.float32),
            pltpu.VMEM((XA_ROWS, BLK), jnp.float32),
            pltpu.VMEM((XP_ROWS, BLK), jnp.float32),
            pltpu.VMEM((PG, LOUT + 2, BLK), jnp.float32),
        ],
        compiler_params=pltpu.CompilerParams(needs_layout_passes=False,
                                             use_tc_tiling_on_sc=True),
    )
    out = run(x.T, W.reshape(-1))
    return jnp.transpose(out, (2, 0, 1))


# speedup vs baseline: 49.7375x; 1.3695x over previous
import jax
import jax.numpy as jnp
import numpy as np
from jax import lax
from jax.experimental import pallas as pl
from jax.experimental.pallas import tpu as pltpu
from jax.experimental.pallas import tpu_sc as plsc

AA = 20
MP = 15
F = 3
NP = 34
LOUT = AA + F - 1
XC = 1 + MP * AA + NP
LANES = 16
NWORK = 32
BLK = 128
NCH = BLK // LANES
XA_ROWS = 184
XP_OFF = 296
XP_ROWS = 39
GROUPS = ((0, 12), (12, 12), (24, 10))
GMAX = 12

_P2J = {0: [0], 1: [1, 2], 2: [0, 1], 3: [2], 4: [1], 6: [2, 3], 7: [3],
        10: [4], 12: [5], 14: [6, 7], 15: [7], 17: [8], 18: [5, 6], 19: [7],
        21: [8], 22: [7, 8], 24: [8], 25: [6], 27: [4], 28: [3], 30: [2],
        31: [1], 33: [0]}


def _body(xh, wh, oh, wv, xa0, xa1, xp, ovA, ovB, si0, si1, soA, soB, nblk):
    cid = lax.axis_index("c")
    sid = lax.axis_index("s")
    wid = sid * 2 + cid
    npair = nblk // 2

    pltpu.sync_copy(wh, wv)

    zero = jnp.zeros((LANES,), jnp.float32)

    def hbase(i):
        return (wid * nblk + i) * BLK

    def xa_slice(i):
        return xh.at[pl.ds(0, XA_ROWS), pl.ds(hbase(i), BLK)]

    def start_in(i, buf, sem):
        pltpu.async_copy(xa_slice(i), buf, sem)

    def wait_in(i, buf, sem):
        pltpu.make_async_copy(xa_slice(i), buf, sem).wait()

    def out_pair(i, g, ov):
        p0, n = GROUPS[g]
        return (ov.at[pl.ds(0, n), pl.ds(0, LOUT), :],
                oh.at[pl.ds(p0, n), :, pl.ds(hbase(i), BLK)])

    def start_out(i, g, ov, sem):
        src, dst = out_pair(i, g, ov)
        pltpu.async_copy(src, dst, sem)

    def wait_out(i, g, ov, sem):
        src, dst = out_pair(i, g, ov)
        pltpu.make_async_copy(src, dst, sem).wait()

    def group_compute(g, ov, xa):
        p0, n = GROUPS[g]

        def chunk(c, carry):
            off = c * LANES
            pep = {}

            def pvec(j, a):
                if (j, a) not in pep:
                    pep[(j, a)] = xa[1 + j * AA + a, pl.ds(off, LANES)]
                return pep[(j, a)]

            aggs = {}
            for p in range(p0, p0 + n):
                js = _P2J.get(p)
                if js is None or tuple(js) in aggs:
                    continue
                if len(js) == 1:
                    aggs[tuple(js)] = [pvec(js[0], a) for a in range(AA)]
                else:
                    aggs[tuple(js)] = [pvec(js[0], a) + pvec(js[1], a)
                                       for a in range(AA)]
            for p in range(p0, p0 + n):
                js = _P2J.get(p)
                if js is None:
                    for l in range(LOUT):
                        ov[p - p0, l, pl.ds(off, LANES)] = zero
                    continue
                agg = aggs[tuple(js)]
                fb = (xp[301 - XP_OFF + p, pl.ds(off, LANES)]
                      .astype(jnp.int32) * F)
                f = [plsc.load_gather(wv, [fb + t]) for t in range(F)]
                for l in range(LOUT):
                    acc = None
                    for t in range(F):
                        a = l - t
                        if 0 <= a < AA:
                            term = f[t] * agg[a]
                            acc = term if acc is None else acc + term
                    ov[p - p0, l, pl.ds(off, LANES)] = acc
            return carry

        lax.fori_loop(0, NCH, chunk, 0)

    start_in(0, xa0, si0)

    def pair(j, carry):
        e = 2 * j
        o = e + 1
        start_in(o, xa1, si1)
        pltpu.sync_copy(xh.at[pl.ds(XP_OFF, XP_ROWS), pl.ds(hbase(e), BLK)],
                        xp)
        wait_in(e, xa0, si0)

        @pl.when(j > 0)
        def _():
            wait_out(e - 1, 1, ovA, soA)
            wait_out(e - 1, 2, ovB, soB)

        group_compute(0, ovA, xa0)
        start_out(e, 0, ovA, soA)
        group_compute(1, ovB, xa0)
        start_out(e, 1, ovB, soB)
        wait_out(e, 0, ovA, soA)
        group_compute(2, ovA, xa0)
        start_out(e, 2, ovA, soA)
        @pl.when(j + 1 < npair)
        def _():
            start_in(e + 2, xa0, si0)
        pltpu.sync_copy(xh.at[pl.ds(XP_OFF, XP_ROWS), pl.ds(hbase(o), BLK)],
                        xp)
        wait_in(o, xa1, si1)
        wait_out(e, 1, ovB, soB)
        group_compute(0, ovB, xa1)
        start_out(o, 0, ovB, soB)
        wait_out(e, 2, ovA, soA)
        group_compute(1, ovA, xa1)
        start_out(o, 1, ovA, soA)
        wait_out(o, 0, ovB, soB)
        group_compute(2, ovB, xa1)
        start_out(o, 2, ovB, soB)
        return carry

    lax.fori_loop(0, npair, pair, 0)
    wait_out(nblk - 1, 1, ovA, soA)
    wait_out(nblk - 1, 2, ovB, soB)


def kernel(x, W):
    B = x.shape[0]
    nblk = B // (NWORK * BLK)
    mesh = plsc.VectorSubcoreMesh(core_axis_name="c", subcore_axis_name="s")
    run = pl.kernel(
        lambda xh, wh, oh, wv, xa0, xa1, xp, ovA, ovB, si0, si1, soA, soB:
            _body(xh, wh, oh, wv, xa0, xa1, xp, ovA, ovB,
                  si0, si1, soA, soB, nblk),
        out_type=jax.ShapeDtypeStruct((NP, LOUT, B), jnp.float32),
        mesh=mesh,
        scratch_types=[
            pltpu.VMEM((AA * F,), jnp.float32),
            pltpu.VMEM((XA_ROWS, BLK), jnp.float32),
            pltpu.VMEM((XA_ROWS, BLK), jnp.float32),
            pltpu.VMEM((XP_ROWS, BLK), jnp.float32),
            pltpu.VMEM((GMAX, LOUT + 2, BLK), jnp.float32),
            pltpu.VMEM((GMAX, LOUT + 2, BLK), jnp.float32),
            pltpu.SemaphoreType.DMA,
            pltpu.SemaphoreType.DMA,
            pltpu.SemaphoreType.DMA,
            pltpu.SemaphoreType.DMA,
        ],
        compiler_params=pltpu.CompilerParams(needs_layout_passes=False,
                                             use_tc_tiling_on_sc=True),
    )
    out = run(x.T, W.reshape(-1))
    return jnp.transpose(out, (2, 0, 1))
